# baseline (speedup 1.0000x reference)
import functools
import jax
import jax.numpy as jnp
from jax import lax
from jax.experimental import pallas as pl
from jax.experimental.pallas import tpu as pltpu
from jax.experimental.pallas import tpu_sc as plsc

NC, NS, L = 2, 16, 16
NW = NC * NS
N_COL = 2048
D = 8
CH = 16
NCV = N_COL // L
R = 8


def make_sc_kernel(n_rows):
    rpw = n_rows // NW
    nch = rpw // CH
    assert rpw % CH == 0
    mesh = plsc.VectorSubcoreMesh(core_axis_name="c", subcore_axis_name="s")

    @functools.partial(
        pl.kernel,
        out_type=jax.ShapeDtypeStruct((n_rows, N_COL), jnp.float32),
        mesh=mesh,
        scratch_types=[
            pltpu.VMEM((D, N_COL), jnp.float32),
            pltpu.VMEM((rpw * D * L,), jnp.float32),
            pltpu.VMEM((2, CH, N_COL), jnp.float32),
            pltpu.SemaphoreType.DMA,
            pltpu.SemaphoreType.DMA,
            pltpu.SemaphoreType.DMA,
        ],
    )
    def sc_rbf(xrep_hbm, yt_hbm, out_hbm, yt_v, xs_v, ob_v, sem0, sem1, semi):
        wid = lax.axis_index("s") * NC + lax.axis_index("c")
        base = wid * rpw
        pltpu.async_copy(yt_hbm, yt_v, semi).wait()
        pltpu.async_copy(xrep_hbm.at[pl.ds(base * D * L, rpw * D * L)], xs_v, semi).wait()
        sems = [sem0, sem1]
        descs = [None, None]
        for ch in range(nch):
            b = ch % 2
            if descs[b] is not None:
                descs[b].wait()

            def group_body(g, _):
                rr0 = g * R
                r0 = ch * CH + rr0
                xv = [[xs_v[pl.ds(((r0 + i) * D + d) * L, L)]
                       for d in range(D)] for i in range(R)]

                def col_body(c, _c):
                    cs = c * L
                    yv = [yt_v[d, pl.ds(cs, L)] for d in range(D)]
                    for i in range(R):
                        acc = None
                        for d in range(D):
                            t = yv[d] - xv[i][d]
                            t = t * t
                            acc = t if acc is None else acc + t
                        ob_v[b, rr0 + i, pl.ds(cs, L)] = jnp.exp(acc * -0.5)
                    return _c

                return lax.fori_loop(0, NCV, col_body, _)

            lax.fori_loop(0, CH // R, group_body, 0)
            descs[b] = pltpu.async_copy(
                ob_v.at[b], out_hbm.at[pl.ds(base + ch * CH, CH)], sems[b])
        for dsc in descs:
            if dsc is not None:
                dsc.wait()

    return sc_rbf


def sc_kernel(x, y):
    n_rows = x.shape[0]
    yt = y.T
    xrep = jnp.broadcast_to(x[:, :, None], (n_rows, D, L)).reshape(n_rows * D * L)
    return make_sc_kernel(n_rows)(xrep, yt)


def kernel(x, y):
    return sc_kernel(x, y)


# speedup vs baseline: 9.5054x; 9.5054x over previous
import jax
import jax.numpy as jnp
from jax.experimental import pallas as pl

BM = 512


def _rbf_block(x_ref, yt_ref, o_ref):
    xb = x_ref[...]
    yb = yt_ref[...]
    x_hi = xb.astype(jnp.bfloat16)
    x_lo = (xb - x_hi.astype(jnp.float32)).astype(jnp.bfloat16)
    y_hi = yb.astype(jnp.bfloat16)
    y_lo = (yb - y_hi.astype(jnp.float32)).astype(jnp.bfloat16)
    lhs = jnp.concatenate([x_hi, x_hi, x_lo], axis=1)
    rhs = jnp.concatenate([y_hi, y_lo, y_hi], axis=0)
    z = jnp.dot(lhs, rhs, preferred_element_type=jnp.float32)
    xn = jnp.sum(xb * xb, axis=1, keepdims=True)
    yn = jnp.sum(yb * yb, axis=0, keepdims=True)
    o_ref[...] = jnp.exp(z - 0.5 * (xn + yn))


def kernel(x, y):
    n_row, d = x.shape
    n_col = y.shape[0]
    yt = y.T
    grid = (n_row // BM,)
    return pl.pallas_call(
        _rbf_block,
        grid=grid,
        in_specs=[
            pl.BlockSpec((BM, d), lambda i: (i, 0)),
            pl.BlockSpec((d, n_col), lambda i: (0, 0)),
        ],
        out_specs=pl.BlockSpec((BM, n_col), lambda i: (i, 0)),
        out_shape=jax.ShapeDtypeStruct((n_row, n_col), jnp.float32),
    )(x, yt)
